# baseline (speedup 1.0000x reference)
import jax
import jax.numpy as jnp
from jax.experimental import pallas as pl
from jax.experimental.pallas import tpu as pltpu

_K = 1024
_D = 64
_N = 16384
_R = 512
_BETA = 0.25


def _vq_block(z_ref, e_ref, dist_ref, idx_ref, onehot_ref, zq_ref,
              loss_ref, perp_ref, loss_acc, cnt_acc, esq_ref, e2_ref):
    i = pl.program_id(0)
    nb = pl.num_programs(0)

    @pl.when(i == 0)
    def _precompute():
        e0 = e_ref[...]
        esq_ref[...] = jnp.sum(e0 * e0, axis=1)[None, :]
        e2_ref[...] = e0 * 2.0

    zt = z_ref[0]
    zb = jnp.transpose(zt)
    e = e_ref[...]

    zsq = jnp.sum(zb * zb, axis=1, keepdims=True)
    dots2 = jax.lax.dot_general(
        zb, e2_ref[...], (((1,), (1,)), ((), ())),
        preferred_element_type=jnp.float32)
    d = (zsq + esq_ref[...]) - dots2
    dist_ref[...] = d

    idxs = jnp.argmin(d, axis=1).astype(jnp.int32)
    idx_ref[...] = idxs.reshape(1, 1, _R)

    iota = jax.lax.broadcasted_iota(jnp.int32, (_R, _K), 1)
    onehot = (iota == idxs[:, None]).astype(jnp.float32)
    onehot_ref[...] = onehot

    zq = jax.lax.dot_general(
        onehot, e, (((1,), (0,)), ((), ())),
        preferred_element_type=jnp.float32)
    zq_ref[...] = jnp.transpose(zq)[None]

    se = jnp.sum((zq - zb) ** 2).reshape(1, 1)
    ones_row = jnp.ones((1, _R), dtype=jnp.float32)
    cnt = jax.lax.dot_general(
        ones_row, onehot, (((1,), (0,)), ((), ())),
        preferred_element_type=jnp.float32)

    @pl.when(i == 0)
    def _init():
        loss_acc[...] = se
        cnt_acc[...] = cnt

    @pl.when(i > 0)
    def _accum():
        loss_acc[...] += se
        cnt_acc[...] += cnt

    @pl.when(i == nb - 1)
    def _finalize():
        loss_ref[...] = (_BETA / (_N * _D)) * loss_acc[...]
        p = cnt_acc[...] * (1.0 / _N)
        ent = jnp.sum(p * jnp.log(p + 1e-10)).reshape(1, 1)
        perp_ref[...] = jnp.exp(-ent)


def kernel(z, embedding):
    b, d, h, w = z.shape
    z3 = z.reshape(b, d, h * w)
    nb = _N // _R

    dist, idx3, onehot, zq3, loss11, perp11 = pl.pallas_call(
        _vq_block,
        grid=(nb,),
        in_specs=[
            pl.BlockSpec((1, _D, _R), lambda i: (i // 2, 0, i % 2)),
            pl.BlockSpec((_K, _D), lambda i: (0, 0)),
        ],
        out_specs=[
            pl.BlockSpec((_R, _K), lambda i: (i, 0)),
            pl.BlockSpec((1, 1, _R), lambda i: (i, 0, 0)),
            pl.BlockSpec((_R, _K), lambda i: (i, 0)),
            pl.BlockSpec((1, _D, _R), lambda i: (i // 2, 0, i % 2)),
            pl.BlockSpec((1, 1), lambda i: (0, 0)),
            pl.BlockSpec((1, 1), lambda i: (0, 0)),
        ],
        out_shape=[
            jax.ShapeDtypeStruct((_N, _K), jnp.float32),
            jax.ShapeDtypeStruct((nb, 1, _R), jnp.int32),
            jax.ShapeDtypeStruct((_N, _K), jnp.float32),
            jax.ShapeDtypeStruct((b, _D, h * w), jnp.float32),
            jax.ShapeDtypeStruct((1, 1), jnp.float32),
            jax.ShapeDtypeStruct((1, 1), jnp.float32),
        ],
        scratch_shapes=[
            pltpu.VMEM((1, 1), jnp.float32),
            pltpu.VMEM((1, _K), jnp.float32),
            pltpu.VMEM((1, _K), jnp.float32),
            pltpu.VMEM((_K, _D), jnp.float32),
        ],
    )(z3, embedding)

    encoding_indices = idx3.reshape(-1)
    z_quantized = zq3.reshape(b, d, h, w)
    loss = loss11[0, 0]
    perplexity = perp11[0, 0]
    return (z_quantized, loss, perplexity, onehot, encoding_indices, dist)


# speedup vs baseline: 1.2081x; 1.2081x over previous
import jax
import jax.numpy as jnp
from jax.experimental import pallas as pl
from jax.experimental.pallas import tpu as pltpu

_K = 1024
_D = 64
_N = 16384
_R = 512
_BETA = 0.25


def _vq_block(z_ref, e_ref, dist_ref, idx_ref, onehot_ref, zq_ref,
              loss_ref, perp_ref, loss_acc, cnt_acc, esq_ref, e2_ref):
    i = pl.program_id(0)
    nb = pl.num_programs(0)

    @pl.when(i == 0)
    def _precompute():
        e0 = e_ref[...]
        esq_ref[...] = jnp.sum(e0 * e0, axis=1)[None, :]
        e2_ref[...] = e0 * 2.0

    zb = z_ref[...]
    e = e_ref[...]

    zsq = jnp.sum(zb * zb, axis=1, keepdims=True)
    dots2 = jax.lax.dot_general(
        zb, e2_ref[...], (((1,), (1,)), ((), ())),
        preferred_element_type=jnp.float32)
    d = (zsq + esq_ref[...]) - dots2
    dist_ref[...] = d

    idxs = jnp.argmin(d, axis=1).astype(jnp.int32)
    idx_ref[...] = idxs.reshape(1, 1, _R)

    iota = jax.lax.broadcasted_iota(jnp.int32, (_R, _K), 1)
    onehot = (iota == idxs[:, None]).astype(jnp.float32)
    onehot_ref[...] = onehot

    zq = jax.lax.dot_general(
        onehot, e, (((1,), (0,)), ((), ())),
        preferred_element_type=jnp.float32)
    zq_ref[...] = zq

    se = jnp.sum((zq - zb) ** 2).reshape(1, 1)
    ones_row = jnp.ones((1, _R), dtype=jnp.float32)
    cnt = jax.lax.dot_general(
        ones_row, onehot, (((1,), (0,)), ((), ())),
        preferred_element_type=jnp.float32)

    @pl.when(i == 0)
    def _init():
        loss_acc[...] = se
        cnt_acc[...] = cnt

    @pl.when(i > 0)
    def _accum():
        loss_acc[...] += se
        cnt_acc[...] += cnt

    @pl.when(i == nb - 1)
    def _finalize():
        loss_ref[...] = (_BETA / (_N * _D)) * loss_acc[...]
        p = cnt_acc[...] * (1.0 / _N)
        ent = jnp.sum(p * jnp.log(p + 1e-10)).reshape(1, 1)
        perp_ref[...] = jnp.exp(-ent)


def kernel(z, embedding):
    b, d, h, w = z.shape
    z_flat = jnp.transpose(z, (0, 2, 3, 1)).reshape(-1, d)
    nb = _N // _R

    dist, idx3, onehot, zq_flat, loss11, perp11 = pl.pallas_call(
        _vq_block,
        grid=(nb,),
        in_specs=[
            pl.BlockSpec((_R, _D), lambda i: (i, 0)),
            pl.BlockSpec((_K, _D), lambda i: (0, 0)),
        ],
        out_specs=[
            pl.BlockSpec((_R, _K), lambda i: (i, 0)),
            pl.BlockSpec((1, 1, _R), lambda i: (i, 0, 0)),
            pl.BlockSpec((_R, _K), lambda i: (i, 0)),
            pl.BlockSpec((_R, _D), lambda i: (i, 0)),
            pl.BlockSpec((1, 1), lambda i: (0, 0)),
            pl.BlockSpec((1, 1), lambda i: (0, 0)),
        ],
        out_shape=[
            jax.ShapeDtypeStruct((_N, _K), jnp.float32),
            jax.ShapeDtypeStruct((nb, 1, _R), jnp.int32),
            jax.ShapeDtypeStruct((_N, _K), jnp.float32),
            jax.ShapeDtypeStruct((_N, _D), jnp.float32),
            jax.ShapeDtypeStruct((1, 1), jnp.float32),
            jax.ShapeDtypeStruct((1, 1), jnp.float32),
        ],
        scratch_shapes=[
            pltpu.VMEM((1, 1), jnp.float32),
            pltpu.VMEM((1, _K), jnp.float32),
            pltpu.VMEM((1, _K), jnp.float32),
            pltpu.VMEM((_K, _D), jnp.float32),
        ],
    )(z_flat, embedding)

    encoding_indices = idx3.reshape(-1)
    z_quantized = jnp.transpose(zq_flat.reshape(b, h, w, d), (0, 3, 1, 2))
    loss = loss11[0, 0]
    perplexity = perp11[0, 0]
    return (z_quantized, loss, perplexity, onehot, encoding_indices, dist)
